# baseline (speedup 1.0000x reference)
import functools

import jax
import jax.numpy as jnp
from jax import lax
from jax.experimental import pallas as pl
from jax.experimental.pallas import tpu as pltpu
from jax.experimental.pallas import tpu_sc as plsc

_NC = 2
_NS = 16
_NW = _NC * _NS

_BS = 512
_LT = 8


def _lookup_call(n_b, n_l, n_v, n_d):
    n_lt = n_l // _LT
    n_pairs = (n_lt - 1) // 2
    mesh = plsc.VectorSubcoreMesh(
        core_axis_name="c", subcore_axis_name="s",
        num_cores=_NC, num_subcores=_NS,
    )

    @functools.partial(
        pl.kernel,
        out_type=jax.ShapeDtypeStruct((n_d, n_l, n_b), jnp.float32),
        mesh=mesh,
        scratch_types=[
            pltpu.VMEM((112,), jnp.float32),
            pltpu.VMEM((n_d * 16 * n_v,), jnp.float32),
            pltpu.VMEM((2, _LT, _BS), jnp.int32),
            pltpu.VMEM((2, n_d, _LT, _BS), jnp.float32),
            pltpu.SemaphoreType.DMA,
            pltpu.SemaphoreType.DMA,
            pltpu.SemaphoreType.DMA,
            pltpu.SemaphoreType.DMA,
        ],
        compiler_params=pltpu.CompilerParams(
            use_tc_tiling_on_sc=True, needs_layout_passes=False,
        ),
    )
    def k(idx_hbm, table_hbm, out_hbm, tab_v, rep_v, slab2_v, stage2_v,
          si_a, si_b, so_a, so_b):
        wid = lax.axis_index("s") * _NC + lax.axis_index("c")
        b0 = wid * _BS
        pltpu.sync_copy(table_hbm, tab_v)
        lanes = lax.iota(jnp.int32, 16)
        zf = jnp.zeros((16,), jnp.float32)
        ws = [tab_v[pl.ds(16 * i, 16)] for i in range(7)]
        for dd in range(n_d):
            for row in range(n_v):
                e = row * n_d + dd
                val = ws[e // 16][e % 16]
                rep_v[pl.ds(dd * 16 * n_v + row * 16, 16)] = zf + val

        sin = [si_a, si_b]
        sout = [so_a, so_b]

        def start_in(buf, lt):
            pltpu.async_copy(
                idx_hbm.at[pl.ds(lt * _LT, _LT), pl.ds(b0, _BS)],
                slab2_v.at[buf], sin[buf])

        def wait_in(buf):
            pltpu.make_async_copy(
                idx_hbm.at[pl.ds(0, _LT), pl.ds(b0, _BS)],
                slab2_v.at[buf], sin[buf]).wait()

        def start_out(buf, lt):
            pltpu.async_copy(
                stage2_v.at[buf],
                out_hbm.at[:, pl.ds(lt * _LT, _LT), pl.ds(b0, _BS)],
                sout[buf])

        def wait_out(buf):
            pltpu.make_async_copy(
                stage2_v.at[buf],
                out_hbm.at[:, pl.ds(0, _LT), pl.ds(b0, _BS)],
                sout[buf]).wait()

        def addr_of(buf, lr, bv):
            bidx = slab2_v[buf, lr, pl.ds(bv * 16, 16)]
            return (bidx << 4) | lanes

        def gathers(addr):
            return [
                plsc.load_gather(
                    rep_v.at[pl.ds(dd * 16 * n_v, 16 * n_v)], [addr])
                for dd in range(n_d)
            ]

        def compute(buf):
            def bv_body(bv):
                prev = gathers(addr_of(buf, 0, bv))
                for lr in range(1, _LT + 1):
                    nxt = []
                    addr = addr_of(buf, lr, bv) if lr < _LT else None
                    for dd in range(n_d):
                        if addr is not None:
                            nxt.append(plsc.load_gather(
                                rep_v.at[pl.ds(dd * 16 * n_v, 16 * n_v)],
                                [addr]))
                        stage2_v[buf, dd, lr - 1, pl.ds(bv * 16, 16)] = prev[dd]
                    prev = nxt

            def bv_pair(i, carry2):
                bv_body(2 * i)
                bv_body(2 * i + 1)
                return carry2

            lax.fori_loop(0, _BS // 32, bv_pair, 0)

        start_in(0, 0)

        def pair_body(t, carry):
            lt_a = 2 * t
            start_in(1, lt_a + 1)
            wait_in(0)

            @pl.when(t > 0)
            def _():
                wait_out(0)

            compute(0)
            start_out(0, lt_a)

            start_in(0, lt_a + 2)
            wait_in(1)

            @pl.when(t > 0)
            def _():
                wait_out(1)

            compute(1)
            start_out(1, lt_a + 1)
            return carry

        lax.fori_loop(0, n_pairs, pair_body, 0)

        wait_in(0)
        wait_out(0)
        compute(0)
        start_out(0, n_lt - 1)
        wait_out(0)
        wait_out(1)

    return k


def kernel(indices, table):
    b, l = indices.shape
    v, d = table.shape
    idx_t = jnp.transpose(indices)
    table_flat = jnp.pad(table.reshape(v * d), (0, 112 - v * d))
    out3 = _lookup_call(b, l, v, d)(idx_t, table_flat)
    return jnp.transpose(out3, (2, 1, 0))


# speedup vs baseline: 1.3939x; 1.3939x over previous
import functools

import jax
import jax.numpy as jnp
from jax import lax
from jax.experimental import pallas as pl
from jax.experimental.pallas import tpu as pltpu
from jax.experimental.pallas import tpu_sc as plsc

_NC = 2
_NS = 16
_NW = _NC * _NS

_BS = 512
_LT = 8


def _lookup_call(n_b, n_l, n_v, n_d):
    n_lt = n_l // _LT
    n_pairs = (n_lt - 1) // 2
    mesh = plsc.VectorSubcoreMesh(
        core_axis_name="c", subcore_axis_name="s",
        num_cores=_NC, num_subcores=_NS,
    )

    @functools.partial(
        pl.kernel,
        out_type=jax.ShapeDtypeStruct((n_d, n_l, n_b), jnp.float32),
        mesh=mesh,
        scratch_types=[
            pltpu.VMEM((112,), jnp.float32),
            pltpu.VMEM((n_d * 16 * n_v,), jnp.float32),
            pltpu.VMEM((2, _LT, _BS), jnp.int32),
            pltpu.VMEM((2, n_d, _LT, _BS), jnp.float32),
            pltpu.SemaphoreType.DMA,
            pltpu.SemaphoreType.DMA,
            pltpu.SemaphoreType.DMA,
            pltpu.SemaphoreType.DMA,
        ],
        compiler_params=pltpu.CompilerParams(
            use_tc_tiling_on_sc=True, needs_layout_passes=False,
        ),
    )
    def k(idx_hbm, table_hbm, out_hbm, tab_v, rep_v, slab2_v, stage2_v,
          si_a, si_b, so_a, so_b):
        wid = lax.axis_index("s") * _NC + lax.axis_index("c")
        b0 = wid * _BS
        pltpu.sync_copy(table_hbm, tab_v)
        lanes = lax.iota(jnp.int32, 16)
        zf = jnp.zeros((16,), jnp.float32)
        ws = [tab_v[pl.ds(16 * i, 16)] for i in range(7)]
        for dd in range(n_d):
            for row in range(n_v):
                e = row * n_d + dd
                val = ws[e // 16][e % 16]
                rep_v[pl.ds(dd * 16 * n_v + row * 16, 16)] = zf + val

        sin = [si_a, si_b]
        sout = [so_a, so_b]

        def start_in(buf, lt):
            pltpu.async_copy(
                idx_hbm.at[pl.ds(lt * _LT, _LT), pl.ds(b0, _BS)],
                slab2_v.at[buf], sin[buf])

        def wait_in(buf):
            pltpu.make_async_copy(
                idx_hbm.at[pl.ds(0, _LT), pl.ds(b0, _BS)],
                slab2_v.at[buf], sin[buf]).wait()

        def start_out(buf, lt):
            pltpu.async_copy(
                stage2_v.at[buf],
                out_hbm.at[:, pl.ds(lt * _LT, _LT), pl.ds(b0, _BS)],
                sout[buf])

        def wait_out(buf):
            pltpu.make_async_copy(
                stage2_v.at[buf],
                out_hbm.at[:, pl.ds(0, _LT), pl.ds(b0, _BS)],
                sout[buf]).wait()

        def addr_of(buf, lr, bv):
            bidx = slab2_v[buf, lr, pl.ds(bv * 16, 16)]
            return (bidx << 4) | lanes

        def gathers(addr):
            return [
                plsc.load_gather(
                    rep_v.at[pl.ds(dd * 16 * n_v, 16 * n_v)], [addr])
                for dd in range(n_d)
            ]

        def compute(buf):
            def bv_body(bv, carry2):
                prev = gathers(addr_of(buf, 0, bv))
                for lr in range(1, _LT + 1):
                    nxt = []
                    addr = addr_of(buf, lr, bv) if lr < _LT else None
                    for dd in range(n_d):
                        if addr is not None:
                            nxt.append(plsc.load_gather(
                                rep_v.at[pl.ds(dd * 16 * n_v, 16 * n_v)],
                                [addr]))
                        stage2_v[buf, dd, lr - 1, pl.ds(bv * 16, 16)] = prev[dd]
                    prev = nxt
                return carry2

            lax.fori_loop(0, _BS // 16, bv_body, 0)

        start_in(0, 0)

        def pair_body(t, carry):
            lt_a = 2 * t
            start_in(1, lt_a + 1)
            wait_in(0)

            @pl.when(t > 0)
            def _():
                wait_out(0)

            compute(0)
            start_out(0, lt_a)

            start_in(0, lt_a + 2)
            wait_in(1)

            @pl.when(t > 0)
            def _():
                wait_out(1)

            compute(1)
            start_out(1, lt_a + 1)
            return carry

        lax.fori_loop(0, n_pairs, pair_body, 0)

        wait_in(0)
        wait_out(0)
        compute(0)
        start_out(0, n_lt - 1)
        wait_out(0)
        wait_out(1)

    return k


def kernel(indices, table):
    b, l = indices.shape
    v, d = table.shape
    idx_t = jnp.transpose(indices)
    table_flat = jnp.pad(table.reshape(v * d), (0, 112 - v * d))
    out3 = _lookup_call(b, l, v, d)(idx_t, table_flat)
    return jnp.transpose(out3, (2, 1, 0))
